# baseline (speedup 1.0000x reference)
import functools
import math

import jax
import jax.numpy as jnp
from jax import lax
from jax.experimental import pallas as pl
from jax.experimental.pallas import tpu as pltpu
from jax.experimental.pallas import tpu_sc as plsc

_D = 64
_SCALE = math.sqrt(_D)
_CHUNK = 128
_LANES = 16


@functools.cache
def _build_sc_lookup(B, n_workers, n_chunks):
    chunk = _CHUNK
    b_per_w = n_chunks * chunk
    info = plsc.get_sparse_core_info()
    nc = info.num_cores

    mesh = plsc.VectorSubcoreMesh(core_axis_name="c", subcore_axis_name="s")

    @functools.partial(
        pl.kernel,
        mesh=mesh,
        out_type=jax.ShapeDtypeStruct((B, _D), jnp.float32),
        compiler_params=pltpu.CompilerParams(use_tc_tiling_on_sc=False),
        scratch_types=[
            pltpu.VMEM((n_chunks, chunk), jnp.int32),
            pltpu.VMEM((chunk, _D), jnp.float32),
            pltpu.VMEM((chunk, _D), jnp.float32),
            pltpu.VMEM((chunk, _D), jnp.float32),
            pltpu.VMEM((chunk, _D), jnp.float32),
            pltpu.SemaphoreType.DMA,
            pltpu.SemaphoreType.DMA,
            pltpu.SemaphoreType.DMA,
            pltpu.SemaphoreType.DMA,
        ],
    )
    def k(x_hbm, table_hbm, out_hbm, idx_v, in0, in1, o0, o1, g0, g1, w0, w1):
        ins = (in0, in1)
        outs = (o0, o1)
        gsems = (g0, g1)
        wsems = (w0, w1)

        wid = lax.axis_index("s") * nc + lax.axis_index("c")
        base = wid * b_per_w

        pltpu.sync_copy(x_hbm.at[wid], idx_v)

        def gather(j, b):
            return pltpu.make_async_copy(
                table_hbm.at[idx_v.at[j]], ins[b], gsems[b])

        def writer(j, b):
            return pltpu.make_async_copy(
                outs[b], out_hbm.at[pl.ds(base + j * chunk, chunk)], wsems[b])

        def scale(b):
            def body(r, carry):
                for c in range(_D // _LANES):
                    sl = pl.ds(_LANES * c, _LANES)
                    outs[b][r, sl] = ins[b][r, sl] * _SCALE
                return carry
            lax.fori_loop(0, chunk, body, 0, unroll=4)

        gather(0, 0).start()
        gather(1, 1).start()

        def body(g, carry):
            for b in range(2):
                j = g * 2 + b
                gather(j, b).wait()

                @pl.when(j >= 2)
                def _():
                    writer(j - 2, b).wait()

                scale(b)

                @pl.when(j + 2 < n_chunks)
                def _():
                    gather(j + 2, b).start()

                writer(j, b).start()
            return carry

        lax.fori_loop(0, n_chunks // 2, body, 0)

        writer(n_chunks - 2, 0).wait()
        writer(n_chunks - 1, 1).wait()

    return k


@jax.jit
def kernel(x, table):
    B = x.size
    info = plsc.get_sparse_core_info()
    nw = info.num_cores * info.num_subcores
    n_chunks = B // (nw * _CHUNK)
    xw = x.reshape(nw, n_chunks, _CHUNK).astype(jnp.int32)
    out = _build_sc_lookup(B, nw, n_chunks)(xw, table)
    return out.reshape(x.shape + (_D,))
